# baseline (speedup 1.0000x reference)
import functools

import jax
import jax.numpy as jnp
from jax.experimental import pallas as pl
from jax.experimental.pallas import tpu as pltpu

TOK = 16384
HSD = 2048
CONF = 128
HID2 = 4096
OUT = 1024

BF = jnp.bfloat16
F32 = jnp.float32


def _silu_f32(x):
    return x * jax.nn.sigmoid(x)


def _l1_body(h_ref, u_ref, w1h_ref, w1u_ref, b1_ref, o_ref):
    acc = jnp.dot(h_ref[...], w1h_ref[...], preferred_element_type=F32)
    acc += jnp.dot(u_ref[...], w1u_ref[...], preferred_element_type=F32)
    acc += b1_ref[...]
    o_ref[...] = _silu_f32(acc).astype(BF)


def _layer1(hb, ub, w1h, w1u, b1r, tm=2048, tn=1024):
    grid = (TOK // tm, HID2 // tn)
    return pl.pallas_call(
        _l1_body,
        grid=grid,
        in_specs=[
            pl.BlockSpec((tm, HSD), lambda m, n: (m, 0)),
            pl.BlockSpec((tm, CONF), lambda m, n: (m, 0)),
            pl.BlockSpec((HSD, tn), lambda m, n: (0, n)),
            pl.BlockSpec((CONF, tn), lambda m, n: (0, n)),
            pl.BlockSpec((1, tn), lambda m, n: (0, n)),
        ],
        out_specs=pl.BlockSpec((tm, tn), lambda m, n: (m, n)),
        out_shape=jax.ShapeDtypeStruct((TOK, HID2), BF),
    )(hb, ub, w1h, w1u, b1r)


def _l2_body(x_ref, w_ref, b2_ref, g_ref, be_ref, o_ref, acc_ref):
    k = pl.program_id(1)
    nk = pl.num_programs(1)
    part = jnp.dot(x_ref[...], w_ref[...], preferred_element_type=F32)

    @pl.when(k == 0)
    def _():
        acc_ref[...] = part

    @pl.when(k != 0)
    def _():
        acc_ref[...] += part

    @pl.when(k == nk - 1)
    def _():
        y = acc_ref[...] + b2_ref[...]
        y = _silu_f32(y)
        mu = jnp.mean(y, axis=-1, keepdims=True)
        var = jnp.mean((y - mu) ** 2, axis=-1, keepdims=True)
        y = (y - mu) / jnp.sqrt(var + 1e-5) * g_ref[...] + be_ref[...]
        o_ref[...] = y.astype(BF)


def _layer2(x1, w2, b2r, gr, ber, tm=512, tk=1024):
    grid = (TOK // tm, HID2 // tk)
    return pl.pallas_call(
        _l2_body,
        grid=grid,
        in_specs=[
            pl.BlockSpec((tm, tk), lambda m, k: (m, k)),
            pl.BlockSpec((tk, HID2), lambda m, k: (k, 0)),
            pl.BlockSpec((1, HID2), lambda m, k: (0, 0)),
            pl.BlockSpec((1, HID2), lambda m, k: (0, 0)),
            pl.BlockSpec((1, HID2), lambda m, k: (0, 0)),
        ],
        out_specs=pl.BlockSpec((tm, HID2), lambda m, k: (m, 0)),
        out_shape=jax.ShapeDtypeStruct((TOK, HID2), BF),
        scratch_shapes=[pltpu.VMEM((tm, HID2), F32)],
    )(x1, w2, b2r, gr, ber)


def _l3_body(x_ref, w_ref, b3_ref, o_ref):
    acc = jnp.dot(x_ref[...], w_ref[...], preferred_element_type=F32)
    o_ref[...] = acc + b3_ref[...]


def _layer3(x2, w3, b3r, tm=1024):
    grid = (TOK // tm,)
    return pl.pallas_call(
        _l3_body,
        grid=grid,
        in_specs=[
            pl.BlockSpec((tm, HID2), lambda m: (m, 0)),
            pl.BlockSpec((HID2, OUT), lambda m: (0, 0)),
            pl.BlockSpec((1, OUT), lambda m: (0, 0)),
        ],
        out_specs=pl.BlockSpec((tm, OUT), lambda m: (m, 0)),
        out_shape=jax.ShapeDtypeStruct((TOK, OUT), F32),
    )(x2, w3, b3r)


@jax.jit
def _run(h, u, W1, b1, W2, b2, gamma, beta, W3, b3):
    hb = h.astype(BF)
    ub = u.astype(BF)
    w1h = W1[:HSD].astype(BF)
    w1u = W1[HSD:].astype(BF)
    x1 = _layer1(hb, ub, w1h, w1u, b1.reshape(1, -1))
    x2 = _layer2(x1, W2.astype(BF), b2.reshape(1, -1),
                 gamma.reshape(1, -1), beta.reshape(1, -1))
    out = _layer3(x2, W3.astype(BF), b3.reshape(1, -1))
    return out


def kernel(h, code_emb, u, W1, b1, W2, b2, gamma, beta, W3, b3):
    out = _run(h, u, W1, b1, W2, b2, gamma, beta, W3, b3)
    zero = jnp.array(0.0, dtype=F32)
    return (out, zero, zero, zero, zero)


# speedup vs baseline: 1.0044x; 1.0044x over previous
import jax
import jax.numpy as jnp
from jax.experimental import pallas as pl
from jax.experimental.pallas import tpu as pltpu

TOK = 16384
HSD = 2048
CONF = 128
HID2 = 4096
OUT = 1024

BF = jnp.bfloat16
F32 = jnp.float32


def _silu_f32(x):
    return x * jax.nn.sigmoid(x)


def _l1_body(h_ref, u_ref, w1h_ref, w1u_ref, b1_ref, o_ref):
    hb = h_ref[...].astype(BF)
    ub = u_ref[...].astype(BF)
    acc = jnp.dot(hb, w1h_ref[...], preferred_element_type=F32)
    acc += jnp.dot(ub, w1u_ref[...], preferred_element_type=F32)
    acc += b1_ref[...]
    o_ref[...] = _silu_f32(acc).astype(BF)


def _layer1(h, u, w1h, w1u, b1r, tm=1024, tn=1024):
    grid = (TOK // tm, HID2 // tn)
    return pl.pallas_call(
        _l1_body,
        grid=grid,
        in_specs=[
            pl.BlockSpec((tm, HSD), lambda m, n: (m, 0)),
            pl.BlockSpec((tm, CONF), lambda m, n: (m, 0)),
            pl.BlockSpec((HSD, tn), lambda m, n: (0, n)),
            pl.BlockSpec((CONF, tn), lambda m, n: (0, n)),
            pl.BlockSpec((1, tn), lambda m, n: (0, n)),
        ],
        out_specs=pl.BlockSpec((tm, tn), lambda m, n: (m, n)),
        out_shape=jax.ShapeDtypeStruct((TOK, HID2), BF),
    )(h, u, w1h, w1u, b1r)


def _l23_body(x_ref, w2_ref, b2_ref, g_ref, be_ref, w3_ref, b3_ref,
              o_ref, acc_ref, xn_ref):
    k = pl.program_id(1)
    nk = pl.num_programs(1)
    part = jnp.dot(x_ref[...], w2_ref[...], preferred_element_type=F32)

    @pl.when(k == 0)
    def _():
        acc_ref[...] = part

    @pl.when(k != 0)
    def _():
        acc_ref[...] += part

    @pl.when(k == nk - 1)
    def _():
        acc_ref[...] = _silu_f32(acc_ref[...] + b2_ref[...])
        s1 = jnp.sum(acc_ref[...], axis=-1, keepdims=True)
        s2 = jnp.sum(acc_ref[...] * acc_ref[...], axis=-1, keepdims=True)
        mu = s1 * (1.0 / HID2)
        var = s2 * (1.0 / HID2) - mu * mu
        rs = jax.lax.rsqrt(var + 1e-5)
        xn_ref[...] = (((acc_ref[...] - mu) * rs) * g_ref[...]
                       + be_ref[...]).astype(BF)
        o_ref[...] = (jnp.dot(xn_ref[...], w3_ref[...],
                              preferred_element_type=F32) + b3_ref[...])


def _layer23(x1, w2, b2r, gr, ber, w3, b3r, tm=512, tk=512):
    grid = (TOK // tm, HID2 // tk)
    return pl.pallas_call(
        _l23_body,
        grid=grid,
        in_specs=[
            pl.BlockSpec((tm, tk), lambda m, k: (m, k)),
            pl.BlockSpec((tk, HID2), lambda m, k: (k, 0)),
            pl.BlockSpec((1, HID2), lambda m, k: (0, 0)),
            pl.BlockSpec((1, HID2), lambda m, k: (0, 0)),
            pl.BlockSpec((1, HID2), lambda m, k: (0, 0)),
            pl.BlockSpec((HID2, OUT), lambda m, k: (0, 0)),
            pl.BlockSpec((1, OUT), lambda m, k: (0, 0)),
        ],
        out_specs=pl.BlockSpec((tm, OUT), lambda m, k: (m, 0)),
        out_shape=jax.ShapeDtypeStruct((TOK, OUT), F32),
        scratch_shapes=[pltpu.VMEM((tm, HID2), F32),
                        pltpu.VMEM((tm, HID2), BF)],
    )(x1, w2, b2r, gr, ber, w3, b3r)


@jax.jit
def _run(h, u, W1, b1, W2, b2, gamma, beta, W3, b3):
    w1h = W1[:HSD].astype(BF)
    w1u = W1[HSD:].astype(BF)
    x1 = _layer1(h, u.astype(BF), w1h, w1u, b1.reshape(1, -1))
    out = _layer23(x1, W2.astype(BF), b2.reshape(1, -1),
                   gamma.reshape(1, -1), beta.reshape(1, -1),
                   W3.astype(BF), b3.reshape(1, -1))
    return out


def kernel(h, code_emb, u, W1, b1, W2, b2, gamma, beta, W3, b3):
    out = _run(h, u, W1, b1, W2, b2, gamma, beta, W3, b3)
    zero = jnp.array(0.0, dtype=F32)
    return (out, zero, zero, zero, zero)


# speedup vs baseline: 1.0047x; 1.0003x over previous
import jax
import jax.numpy as jnp
from jax.experimental import pallas as pl
from jax.experimental.pallas import tpu as pltpu

TOK = 16384
HSD = 2048
CONF = 128
HID2 = 4096
OUT = 1024

BF = jnp.bfloat16
F32 = jnp.float32


def _silu_f32(x):
    return x * jax.nn.sigmoid(x)


def _l1_body(h_ref, u_ref, w1h_ref, w1u_ref, b1_ref, o_ref, hb_ref):
    hb_ref[...] = h_ref[...].astype(BF)
    acc = jnp.dot(hb_ref[...], w1h_ref[...], preferred_element_type=F32)
    acc += jnp.dot(u_ref[...].astype(BF), w1u_ref[...],
                   preferred_element_type=F32)
    acc += b1_ref[...]
    o_ref[...] = _silu_f32(acc).astype(BF)


def _layer1(h, u, w1h, w1u, b1r, tm=512):
    grid = (TOK // tm,)
    return pl.pallas_call(
        _l1_body,
        grid=grid,
        in_specs=[
            pl.BlockSpec((tm, HSD), lambda m: (m, 0)),
            pl.BlockSpec((tm, CONF), lambda m: (m, 0)),
            pl.BlockSpec((HSD, HID2), lambda m: (0, 0)),
            pl.BlockSpec((CONF, HID2), lambda m: (0, 0)),
            pl.BlockSpec((1, HID2), lambda m: (0, 0)),
        ],
        out_specs=pl.BlockSpec((tm, HID2), lambda m: (m, 0)),
        out_shape=jax.ShapeDtypeStruct((TOK, HID2), BF),
        scratch_shapes=[pltpu.VMEM((tm, HSD), BF)],
    )(h, u, w1h, w1u, b1r)


def _l23_body(x_ref, w2_ref, b2_ref, g_ref, be_ref, w3_ref, b3_ref,
              o_ref, acc_ref, xn_ref):
    k = pl.program_id(1)
    nk = pl.num_programs(1)
    part = jnp.dot(x_ref[...], w2_ref[...], preferred_element_type=F32)

    @pl.when(k == 0)
    def _():
        acc_ref[...] = part

    @pl.when(k != 0)
    def _():
        acc_ref[...] += part

    @pl.when(k == nk - 1)
    def _():
        acc_ref[...] = _silu_f32(acc_ref[...] + b2_ref[...])
        a = acc_ref[...]
        s1 = jnp.sum(a, axis=-1, keepdims=True)
        s2 = jnp.sum(a * a, axis=-1, keepdims=True)
        mu = s1 * (1.0 / HID2)
        var = s2 * (1.0 / HID2) - mu * mu
        rs = jax.lax.rsqrt(var + 1e-5)
        xn_ref[...] = (((acc_ref[...] - mu) * rs) * g_ref[...]
                       + be_ref[...]).astype(BF)
        o_ref[...] = (jnp.dot(xn_ref[...], w3_ref[...],
                              preferred_element_type=F32) + b3_ref[...])


def _layer23(x1, w2, b2r, gr, ber, w3, b3r, tm=512, tk=512):
    grid = (TOK // tm, HID2 // tk)
    return pl.pallas_call(
        _l23_body,
        grid=grid,
        in_specs=[
            pl.BlockSpec((tm, tk), lambda m, k: (m, k)),
            pl.BlockSpec((tk, HID2), lambda m, k: (k, 0)),
            pl.BlockSpec((1, HID2), lambda m, k: (0, 0)),
            pl.BlockSpec((1, HID2), lambda m, k: (0, 0)),
            pl.BlockSpec((1, HID2), lambda m, k: (0, 0)),
            pl.BlockSpec((HID2, OUT), lambda m, k: (0, 0)),
            pl.BlockSpec((1, OUT), lambda m, k: (0, 0)),
        ],
        out_specs=pl.BlockSpec((tm, OUT), lambda m, k: (m, 0)),
        out_shape=jax.ShapeDtypeStruct((TOK, OUT), F32),
        scratch_shapes=[pltpu.VMEM((tm, HID2), F32),
                        pltpu.VMEM((tm, HID2), BF)],
    )(x1, w2, b2r, gr, ber, w3, b3r)


@jax.jit
def _run(h, u, W1, b1, W2, b2, gamma, beta, W3, b3):
    w1h = W1[:HSD].astype(BF)
    w1u = W1[HSD:].astype(BF)
    x1 = _layer1(h, u, w1h, w1u, b1.reshape(1, -1))
    out = _layer23(x1, W2.astype(BF), b2.reshape(1, -1),
                   gamma.reshape(1, -1), beta.reshape(1, -1),
                   W3.astype(BF), b3.reshape(1, -1))
    return out


def kernel(h, code_emb, u, W1, b1, W2, b2, gamma, beta, W3, b3):
    out = _run(h, u, W1, b1, W2, b2, gamma, beta, W3, b3)
    zero = jnp.array(0.0, dtype=F32)
    return (out, zero, zero, zero, zero)


# speedup vs baseline: 1.2013x; 1.1957x over previous
import jax
import jax.numpy as jnp
from jax.experimental import pallas as pl
from jax.experimental.pallas import tpu as pltpu

TOK = 16384
HSD = 2048
CONF = 128
HID2 = 4096
OUT = 1024

BF = jnp.bfloat16
F32 = jnp.float32


def _silu_f32(x):
    return x * jax.nn.sigmoid(x)


def _l1_body(h_ref, u_ref, w1h_ref, w1u_ref, b1_ref, o_ref, hb_ref):
    hb_ref[...] = h_ref[...].astype(BF)
    acc = jnp.dot(hb_ref[...], w1h_ref[...], preferred_element_type=F32)
    acc += jnp.dot(u_ref[...].astype(BF), w1u_ref[...],
                   preferred_element_type=F32)
    acc += b1_ref[...]
    o_ref[...] = _silu_f32(acc).astype(BF)


def _layer1(h, u, w1h, w1u, b1r, tm=512):
    grid = (TOK // tm,)
    return pl.pallas_call(
        _l1_body,
        grid=grid,
        in_specs=[
            pl.BlockSpec((tm, HSD), lambda m: (m, 0)),
            pl.BlockSpec((tm, CONF), lambda m: (m, 0)),
            pl.BlockSpec((HSD, HID2), lambda m: (0, 0)),
            pl.BlockSpec((CONF, HID2), lambda m: (0, 0)),
            pl.BlockSpec((1, HID2), lambda m: (0, 0)),
        ],
        out_specs=pl.BlockSpec((tm, HID2), lambda m: (m, 0)),
        out_shape=jax.ShapeDtypeStruct((TOK, HID2), BF),
        scratch_shapes=[pltpu.VMEM((tm, HSD), BF)],
    )(h, u, w1h, w1u, b1r)


def _l23_body(x_ref, w2_ref, b2_ref, g_ref, be_ref, w3_ref, b3_ref,
              o_ref, acc_ref, xn_ref):
    acc_ref[...] = _silu_f32(
        jnp.dot(x_ref[...], w2_ref[...], preferred_element_type=F32)
        + b2_ref[...])
    a = acc_ref[...]
    s1 = jnp.sum(a, axis=-1, keepdims=True)
    s2 = jnp.sum(a * a, axis=-1, keepdims=True)
    mu = s1 * (1.0 / HID2)
    var = s2 * (1.0 / HID2) - mu * mu
    rs = jax.lax.rsqrt(var + 1e-5)
    xn_ref[...] = (((acc_ref[...] - mu) * rs) * g_ref[...]
                   + be_ref[...]).astype(BF)
    o_ref[...] = (jnp.dot(xn_ref[...], w3_ref[...],
                          preferred_element_type=F32) + b3_ref[...])


def _layer23(x1, w2, b2r, gr, ber, w3, b3r, tm=128):
    grid = (TOK // tm,)
    return pl.pallas_call(
        _l23_body,
        grid=grid,
        in_specs=[
            pl.BlockSpec((tm, HID2), lambda m: (m, 0)),
            pl.BlockSpec((HID2, HID2), lambda m: (0, 0)),
            pl.BlockSpec((1, HID2), lambda m: (0, 0)),
            pl.BlockSpec((1, HID2), lambda m: (0, 0)),
            pl.BlockSpec((1, HID2), lambda m: (0, 0)),
            pl.BlockSpec((HID2, OUT), lambda m: (0, 0)),
            pl.BlockSpec((1, OUT), lambda m: (0, 0)),
        ],
        out_specs=pl.BlockSpec((tm, OUT), lambda m: (m, 0)),
        out_shape=jax.ShapeDtypeStruct((TOK, OUT), F32),
        scratch_shapes=[pltpu.VMEM((tm, HID2), F32),
                        pltpu.VMEM((tm, HID2), BF)],
    )(x1, w2, b2r, gr, ber, w3, b3r)


@jax.jit
def _run(h, u, W1, b1, W2, b2, gamma, beta, W3, b3):
    w1h = W1[:HSD].astype(BF)
    w1u = W1[HSD:].astype(BF)
    x1 = _layer1(h, u, w1h, w1u, b1.reshape(1, -1))
    out = _layer23(x1, W2.astype(BF), b2.reshape(1, -1),
                   gamma.reshape(1, -1), beta.reshape(1, -1),
                   W3.astype(BF), b3.reshape(1, -1))
    return out


def kernel(h, code_emb, u, W1, b1, W2, b2, gamma, beta, W3, b3):
    out = _run(h, u, W1, b1, W2, b2, gamma, beta, W3, b3)
    zero = jnp.array(0.0, dtype=F32)
    return (out, zero, zero, zero, zero)


# speedup vs baseline: 1.2669x; 1.0545x over previous
import jax
import jax.numpy as jnp
from jax.experimental import pallas as pl
from jax.experimental.pallas import tpu as pltpu

TOK = 16384
HSD = 2048
CONF = 128
HID2 = 4096
OUT = 1024

BF = jnp.bfloat16
F32 = jnp.float32


def _silu_f32(x):
    return x * jax.nn.sigmoid(x)


def _l1_body(h_ref, u_ref, w1h_ref, w1u_ref, b1_ref, o_ref, hb_ref):
    hb_ref[...] = h_ref[...].astype(BF)
    acc = jnp.dot(hb_ref[...], w1h_ref[...], preferred_element_type=F32)
    acc += jnp.dot(u_ref[...].astype(BF), w1u_ref[...],
                   preferred_element_type=F32)
    acc += b1_ref[...]
    o_ref[...] = _silu_f32(acc).astype(BF)


def _layer1(h, u, w1h, w1u, b1r, tm=512):
    grid = (TOK // tm,)
    return pl.pallas_call(
        _l1_body,
        grid=grid,
        in_specs=[
            pl.BlockSpec((tm, HSD), lambda m: (m, 0)),
            pl.BlockSpec((tm, CONF), lambda m: (m, 0)),
            pl.BlockSpec((HSD, HID2), lambda m: (0, 0)),
            pl.BlockSpec((CONF, HID2), lambda m: (0, 0)),
            pl.BlockSpec((1, HID2), lambda m: (0, 0)),
        ],
        out_specs=pl.BlockSpec((tm, HID2), lambda m: (m, 0)),
        out_shape=jax.ShapeDtypeStruct((TOK, HID2), BF),
        scratch_shapes=[pltpu.VMEM((tm, HSD), BF)],
    )(h, u, w1h, w1u, b1r)


def _l23_body(x_ref, w2_ref, b2_ref, g_ref, be_ref, w3_ref, b3_ref,
              o_ref, acc_ref, xn_ref):
    acc_ref[...] = _silu_f32(
        jnp.dot(x_ref[...], w2_ref[...], preferred_element_type=F32)
        + b2_ref[...])
    a = acc_ref[...]
    s1 = jnp.sum(a, axis=-1, keepdims=True)
    s2 = jnp.sum(a * a, axis=-1, keepdims=True)
    mu = s1 * (1.0 / HID2)
    var = s2 * (1.0 / HID2) - mu * mu
    rs = jax.lax.rsqrt(var + 1e-5)
    xn_ref[...] = (((acc_ref[...] - mu) * rs) * g_ref[...]
                   + be_ref[...]).astype(BF)
    o_ref[...] = (jnp.dot(xn_ref[...], w3_ref[...],
                          preferred_element_type=F32) + b3_ref[...])


def _layer23(x1, w2, b2r, gr, ber, w3, b3r, tm=256):
    grid = (TOK // tm,)
    return pl.pallas_call(
        _l23_body,
        grid=grid,
        in_specs=[
            pl.BlockSpec((tm, HID2), lambda m: (m, 0)),
            pl.BlockSpec((HID2, HID2), lambda m: (0, 0)),
            pl.BlockSpec((1, HID2), lambda m: (0, 0)),
            pl.BlockSpec((1, HID2), lambda m: (0, 0)),
            pl.BlockSpec((1, HID2), lambda m: (0, 0)),
            pl.BlockSpec((HID2, OUT), lambda m: (0, 0)),
            pl.BlockSpec((1, OUT), lambda m: (0, 0)),
        ],
        out_specs=pl.BlockSpec((tm, OUT), lambda m: (m, 0)),
        out_shape=jax.ShapeDtypeStruct((TOK, OUT), F32),
        scratch_shapes=[pltpu.VMEM((tm, HID2), F32),
                        pltpu.VMEM((tm, HID2), BF)],
    )(x1, w2, b2r, gr, ber, w3, b3r)


@jax.jit
def _run(h, u, W1, b1, W2, b2, gamma, beta, W3, b3):
    w1h = W1[:HSD].astype(BF)
    w1u = W1[HSD:].astype(BF)
    x1 = _layer1(h, u, w1h, w1u, b1.reshape(1, -1))
    out = _layer23(x1, W2.astype(BF), b2.reshape(1, -1),
                   gamma.reshape(1, -1), beta.reshape(1, -1),
                   W3.astype(BF), b3.reshape(1, -1))
    return out


def kernel(h, code_emb, u, W1, b1, W2, b2, gamma, beta, W3, b3):
    out = _run(h, u, W1, b1, W2, b2, gamma, beta, W3, b3)
    zero = jnp.array(0.0, dtype=F32)
    return (out, zero, zero, zero, zero)
